# baseline (device time: 35941 ns/iter reference)
import jax
import jax.numpy as jnp
from jax import lax
from jax.experimental import pallas as pl
from jax.experimental.pallas import tpu as pltpu

N_DEV = 16
B, SQ, SKV, HQ_LOC, DH = 2, 128, 128, 4, 64
D_MODEL = 512
D_HEADS = HQ_LOC * DH
ROWS = B * SQ
QROWS = ROWS // 4
ZROWS = QROWS // 4


def kernel(x, Wq, K_ext, V_ext, Wo):
    Wq_r = Wq.reshape(D_MODEL, N_DEV, D_HEADS)
    Wo_r = Wo.reshape(N_DEV, D_HEADS, D_MODEL)

    def body(x_ref, wq_hbm_ref, k_ref, v_ref, wo_hbm_ref, out_ref,
             acc_ref, pslab_ref, zslab_ref, wq_vmem, wo_vmem,
             copy_sems, prs_send, prs_recv, zb_send, zb_recv,
             pag_send, pag_recv):
        my_pos = lax.axis_index("i")
        g = my_pos & 3
        zpos = my_pos >> 2
        base = my_pos - g

        def plane_dev(o):
            return base + (g ^ o)

        def z_dev(o):
            return ((zpos ^ o) << 2) + g

        cp_wq = pltpu.make_async_copy(
            wq_hbm_ref.at[:, pl.ds(my_pos, 1), :], wq_vmem, copy_sems.at[0]
        )
        cp_wq.start()
        cp_wo = pltpu.make_async_copy(
            wo_hbm_ref.at[pl.ds(my_pos, 1)], wo_vmem, copy_sems.at[1]
        )
        cp_wo.start()

        barrier_sem = pltpu.get_barrier_semaphore()
        for o in (1, 2, 3):
            for dev in (plane_dev(o), z_dev(o)):
                pl.semaphore_signal(
                    barrier_sem, inc=1,
                    device_id=(dev,), device_id_type=pl.DeviceIdType.MESH,
                )
        pl.semaphore_wait(barrier_sem, 6)

        xm = x_ref[...].reshape(ROWS, D_MODEL)
        cp_wq.wait()
        q = jnp.dot(
            xm, wq_vmem[:, 0, :], preferred_element_type=jnp.float32
        )
        q4 = q.reshape(B, SQ, HQ_LOC, DH)

        ctx_rows = []
        for bb in range(B):
            parts = []
            for h in range(HQ_LOC):
                qb = q4[bb, :, h, :]
                kb = k_ref[bb, :, h, :]
                s = lax.dot_general(
                    qb, kb, (((1,), (1,)), ((), ())),
                    preferred_element_type=jnp.float32,
                ) * 0.125
                m = jnp.max(s, axis=-1, keepdims=True)
                w = jnp.exp(s - m)
                w = w / jnp.sum(w, axis=-1, keepdims=True)
                vb = v_ref[bb, :, h, :]
                parts.append(
                    jnp.dot(w, vb, preferred_element_type=jnp.float32)
                )
            ctx_rows.append(jnp.concatenate(parts, axis=-1))
        ctx = jnp.concatenate(ctx_rows, axis=0)

        cp_wo.wait()
        acc_ref[...] = jnp.dot(
            ctx, wo_vmem[0], preferred_element_type=jnp.float32
        )

        keep_lo = g * QROWS

        prs = []
        for o in (1, 2, 3):
            rdma = pltpu.make_async_remote_copy(
                src_ref=acc_ref.at[pl.ds((g ^ o) * QROWS, QROWS), :],
                dst_ref=pslab_ref.at[o - 1],
                send_sem=prs_send.at[o - 1],
                recv_sem=prs_recv.at[o - 1],
                device_id=(plane_dev(o),),
                device_id_type=pl.DeviceIdType.MESH,
            )
            rdma.start()
            prs.append(rdma)
        for rdma in prs:
            rdma.wait_recv()
        acc_ref[pl.ds(keep_lo, QROWS), :] = (
            acc_ref[pl.ds(keep_lo, QROWS), :]
            + pslab_ref[0] + pslab_ref[1] + pslab_ref[2]
        )
        for rdma in prs:
            rdma.wait_send()

        zb = []
        for o in (1, 2, 3):
            rdma = pltpu.make_async_remote_copy(
                src_ref=acc_ref.at[pl.ds(keep_lo, QROWS), :],
                dst_ref=zslab_ref.at[o - 1],
                send_sem=zb_send.at[o - 1],
                recv_sem=zb_recv.at[o - 1],
                device_id=(z_dev(o),),
                device_id_type=pl.DeviceIdType.MESH,
            )
            rdma.start()
            zb.append(rdma)
        for rdma in zb:
            rdma.wait_recv()
            rdma.wait_send()
        acc_ref[pl.ds(keep_lo, QROWS), :] = (
            acc_ref[pl.ds(keep_lo, QROWS), :]
            + zslab_ref[0] + zslab_ref[1] + zslab_ref[2]
        )

        pag = []
        for o in (1, 2, 3):
            rdma = pltpu.make_async_remote_copy(
                src_ref=acc_ref.at[pl.ds(keep_lo, QROWS), :],
                dst_ref=acc_ref.at[pl.ds(keep_lo, QROWS), :],
                send_sem=pag_send.at[o - 1],
                recv_sem=pag_recv.at[o - 1],
                device_id=(plane_dev(o),),
                device_id_type=pl.DeviceIdType.MESH,
            )
            rdma.start()
            pag.append(rdma)
        for rdma in pag:
            rdma.wait_recv()
            rdma.wait_send()

        out_ref[...] = acc_ref[...].reshape(B, SQ, D_MODEL)

    return pl.pallas_call(
        body,
        out_shape=jax.ShapeDtypeStruct((B, SQ, D_MODEL), jnp.float32),
        in_specs=[
            pl.BlockSpec(memory_space=pltpu.VMEM),
            pl.BlockSpec(memory_space=pl.ANY),
            pl.BlockSpec(memory_space=pltpu.VMEM),
            pl.BlockSpec(memory_space=pltpu.VMEM),
            pl.BlockSpec(memory_space=pl.ANY),
        ],
        out_specs=pl.BlockSpec(memory_space=pltpu.VMEM),
        scratch_shapes=[
            pltpu.VMEM((ROWS, D_MODEL), jnp.float32),
            pltpu.VMEM((3, QROWS, D_MODEL), jnp.float32),
            pltpu.VMEM((3, QROWS, D_MODEL), jnp.float32),
            pltpu.VMEM((D_MODEL, 1, D_HEADS), jnp.float32),
            pltpu.VMEM((1, D_HEADS, D_MODEL), jnp.float32),
            pltpu.SemaphoreType.DMA((2,)),
            pltpu.SemaphoreType.DMA((3,)),
            pltpu.SemaphoreType.DMA((3,)),
            pltpu.SemaphoreType.DMA((3,)),
            pltpu.SemaphoreType.DMA((3,)),
            pltpu.SemaphoreType.DMA((3,)),
            pltpu.SemaphoreType.DMA((3,)),
        ],
        compiler_params=pltpu.CompilerParams(collective_id=0),
    )(x, Wq_r, K_ext, V_ext, Wo_r)


# device time: 30702 ns/iter; 1.1706x vs baseline; 1.1706x over previous
import jax
import jax.numpy as jnp
from jax import lax
from jax.experimental import pallas as pl
from jax.experimental.pallas import tpu as pltpu

N_DEV = 16
B, SQ, SKV, HQ_LOC, DH = 2, 128, 128, 4, 64
D_MODEL = 512
D_HEADS = HQ_LOC * DH
ROWS = B * SQ
QROWS = ROWS // 4
ZROWS = QROWS // 4


def kernel(x, Wq, K_ext, V_ext, Wo):
    my = lax.axis_index("i")
    Wq_my = lax.dynamic_slice_in_dim(Wq, my * D_HEADS, D_HEADS, axis=1)
    Wo_my = lax.dynamic_slice_in_dim(Wo, my * D_HEADS, D_HEADS, axis=0)

    def body(x_ref, wq_ref, k_ref, v_ref, wo_ref, out_ref,
             acc_ref, pslab_ref, zslab_ref,
             prs_send, prs_recv, zrs_send, zrs_recv,
             zag_send, zag_recv, pag_send, pag_recv):
        my_pos = lax.axis_index("i")
        g = my_pos & 3
        zpos = my_pos >> 2
        base = my_pos - g

        def plane_dev(o):
            return base + (g ^ o)

        def z_dev(o):
            return ((zpos ^ o) << 2) + g

        barrier_sem = pltpu.get_barrier_semaphore()
        for o in (1, 2, 3):
            for dev in (plane_dev(o), z_dev(o)):
                pl.semaphore_signal(
                    barrier_sem, inc=1,
                    device_id=(dev,), device_id_type=pl.DeviceIdType.MESH,
                )
        pl.semaphore_wait(barrier_sem, 6)

        xm = x_ref[...].reshape(ROWS, D_MODEL)
        q = jnp.dot(xm, wq_ref[...], preferred_element_type=jnp.float32)
        q4 = q.reshape(B, SQ, HQ_LOC, DH)

        ctx_rows = []
        for bb in range(B):
            parts = []
            for h in range(HQ_LOC):
                qb = q4[bb, :, h, :]
                kb = k_ref[bb, :, h, :]
                s = lax.dot_general(
                    qb, kb, (((1,), (1,)), ((), ())),
                    preferred_element_type=jnp.float32,
                ) * 0.125
                m = jnp.max(s, axis=-1, keepdims=True)
                w = jnp.exp(s - m)
                w = w / jnp.sum(w, axis=-1, keepdims=True)
                vb = v_ref[bb, :, h, :]
                parts.append(
                    jnp.dot(w, vb, preferred_element_type=jnp.float32)
                )
            ctx_rows.append(jnp.concatenate(parts, axis=-1))
        ctx = jnp.concatenate(ctx_rows, axis=0)

        acc_ref[...] = jnp.dot(
            ctx, wo_ref[...], preferred_element_type=jnp.float32
        )

        keep_lo = g * QROWS
        blk_lo = keep_lo + zpos * ZROWS

        prs = []
        for o in (1, 2, 3):
            rdma = pltpu.make_async_remote_copy(
                src_ref=acc_ref.at[pl.ds((g ^ o) * QROWS, QROWS), :],
                dst_ref=pslab_ref.at[o - 1],
                send_sem=prs_send.at[o - 1],
                recv_sem=prs_recv.at[o - 1],
                device_id=(plane_dev(o),),
                device_id_type=pl.DeviceIdType.MESH,
            )
            rdma.start()
            prs.append(rdma)
        for rdma in prs:
            rdma.wait_recv()
        acc_ref[pl.ds(keep_lo, QROWS), :] = (
            acc_ref[pl.ds(keep_lo, QROWS), :]
            + pslab_ref[0] + pslab_ref[1] + pslab_ref[2]
        )
        for rdma in prs:
            rdma.wait_send()

        zrs = []
        for o in (1, 2, 3):
            rdma = pltpu.make_async_remote_copy(
                src_ref=acc_ref.at[pl.ds(keep_lo + (zpos ^ o) * ZROWS, ZROWS), :],
                dst_ref=zslab_ref.at[o - 1],
                send_sem=zrs_send.at[o - 1],
                recv_sem=zrs_recv.at[o - 1],
                device_id=(z_dev(o),),
                device_id_type=pl.DeviceIdType.MESH,
            )
            rdma.start()
            zrs.append(rdma)
        for rdma in zrs:
            rdma.wait_recv()
        acc_ref[pl.ds(blk_lo, ZROWS), :] = (
            acc_ref[pl.ds(blk_lo, ZROWS), :]
            + zslab_ref[0] + zslab_ref[1] + zslab_ref[2]
        )
        for rdma in zrs:
            rdma.wait_send()

        zag = []
        for o in (1, 2, 3):
            rdma = pltpu.make_async_remote_copy(
                src_ref=acc_ref.at[pl.ds(blk_lo, ZROWS), :],
                dst_ref=acc_ref.at[pl.ds(blk_lo, ZROWS), :],
                send_sem=zag_send.at[o - 1],
                recv_sem=zag_recv.at[o - 1],
                device_id=(z_dev(o),),
                device_id_type=pl.DeviceIdType.MESH,
            )
            rdma.start()
            zag.append(rdma)
        for rdma in zag:
            rdma.wait_recv()
            rdma.wait_send()

        pag = []
        for o in (1, 2, 3):
            rdma = pltpu.make_async_remote_copy(
                src_ref=acc_ref.at[pl.ds(keep_lo, QROWS), :],
                dst_ref=acc_ref.at[pl.ds(keep_lo, QROWS), :],
                send_sem=pag_send.at[o - 1],
                recv_sem=pag_recv.at[o - 1],
                device_id=(plane_dev(o),),
                device_id_type=pl.DeviceIdType.MESH,
            )
            rdma.start()
            pag.append(rdma)
        for rdma in pag:
            rdma.wait_recv()
            rdma.wait_send()

        out_ref[...] = acc_ref[...].reshape(B, SQ, D_MODEL)

    return pl.pallas_call(
        body,
        out_shape=jax.ShapeDtypeStruct((B, SQ, D_MODEL), jnp.float32),
        in_specs=[
            pl.BlockSpec(memory_space=pltpu.VMEM),
            pl.BlockSpec(memory_space=pltpu.VMEM),
            pl.BlockSpec(memory_space=pltpu.VMEM),
            pl.BlockSpec(memory_space=pltpu.VMEM),
            pl.BlockSpec(memory_space=pltpu.VMEM),
        ],
        out_specs=pl.BlockSpec(memory_space=pltpu.VMEM),
        scratch_shapes=[
            pltpu.VMEM((ROWS, D_MODEL), jnp.float32),
            pltpu.VMEM((3, QROWS, D_MODEL), jnp.float32),
            pltpu.VMEM((3, ZROWS, D_MODEL), jnp.float32),
            pltpu.SemaphoreType.DMA((3,)),
            pltpu.SemaphoreType.DMA((3,)),
            pltpu.SemaphoreType.DMA((3,)),
            pltpu.SemaphoreType.DMA((3,)),
            pltpu.SemaphoreType.DMA((3,)),
            pltpu.SemaphoreType.DMA((3,)),
            pltpu.SemaphoreType.DMA((3,)),
            pltpu.SemaphoreType.DMA((3,)),
        ],
        compiler_params=pltpu.CompilerParams(collective_id=0),
    )(x, Wq_my, K_ext, V_ext, Wo_my)
